# baseline (speedup 1.0000x reference)
import functools

import jax
import jax.numpy as jnp
from jax import lax
from jax.experimental import pallas as pl
from jax.experimental.pallas import tpu as pltpu
from jax.experimental.pallas import tpu_sc as plsc

_TN = 256
_TK = 912
_PASS = 3


def _rn_bf16(x):
    return x.astype(jnp.bfloat16).astype(jnp.float32)


def _vq_body(scale, z_ref, cb_ref, idx_ref, loss_ref,
             zsq_ref, pv_ref, pi_ref, bval_ref, bidx_ref, bdist_ref,
             lsum_ref):
    i = pl.program_id(0)
    j = pl.program_id(1)
    ni = pl.num_programs(0)
    nj = pl.num_programs(1)

    z = z_ref[...]

    @pl.when(j == 0)
    def _init():
        zsq_ref[...] = jnp.sum(z * z, axis=1)
        bval_ref[...] = jnp.full((z.shape[0],), jnp.inf, jnp.float32)
        bidx_ref[...] = jnp.zeros((z.shape[0],), jnp.int32)
        bdist_ref[...] = jnp.zeros((z.shape[0],), jnp.float32)

    cb = cb_ref[...]
    esq = jnp.sum(cb * cb, axis=1)
    mm = lax.dot_general(z, cb, (((1,), (1,)), ((), ())),
                         preferred_element_type=jnp.float32)
    dist = (zsq_ref[...][:, None] + esq[None, :]) - 2.0 * mm
    m = jnp.min(dist, axis=1)
    tk = dist.shape[1]
    iota = lax.broadcasted_iota(jnp.int32, dist.shape, 1)
    gidx = jnp.min(jnp.where(dist == m[:, None], iota, tk), axis=1) + j * tk

    @pl.when(j % _PASS == 0)
    def _pass_start():
        pv_ref[...] = m
        pi_ref[...] = gidx

    @pl.when(j % _PASS != 0)
    def _pass_cont():
        upd = m < pv_ref[...]
        pi_ref[...] = jnp.where(upd, gidx, pi_ref[...])
        pv_ref[...] = jnp.where(upd, m, pv_ref[...])

    @pl.when(j % _PASS == _PASS - 1)
    def _pass_end():
        pv = pv_ref[...]
        pi = pi_ref[...]
        av = bval_ref[...]
        better = pv < av
        take = better | ((pv == av) & (pi < bidx_ref[...]))
        bidx_ref[...] = jnp.where(take, pi, bidx_ref[...])
        bdist_ref[...] = jnp.where(take, pv, bdist_ref[...])
        bval_ref[...] = jnp.where(better, _rn_bf16(pv), av)

    @pl.when(j == nj - 1)
    def _fin():
        idx_ref[...] = bidx_ref[...]

        @pl.when(i == 0)
        def _z():
            lsum_ref[0] = 0.0

        lsum_ref[0] += jnp.sum(bdist_ref[...])

        @pl.when(i == ni - 1)
        def _w():
            loss_ref[...] = jnp.full((1, 1), lsum_ref[0] * scale, jnp.float32)


def _vq_argmin(z, cb):
    n, d = z.shape
    k = cb.shape[0]
    nj = -(-k // _TK)
    kpad = nj * _TK
    if kpad != k:
        cb = jnp.concatenate(
            [cb, jnp.full((kpad - k, d), 1e4, jnp.float32)], axis=0)
    scale = 1.25 / (n * d)
    idx, loss11 = pl.pallas_call(
        functools.partial(_vq_body, scale),
        grid=(n // _TN, nj),
        in_specs=[
            pl.BlockSpec((_TN, d), lambda i, j: (i, 0)),
            pl.BlockSpec((_TK, d), lambda i, j: (j, 0)),
        ],
        out_specs=[
            pl.BlockSpec((_TN,), lambda i, j: (i,)),
            pl.BlockSpec((1, 1), lambda i, j: (0, 0)),
        ],
        out_shape=[
            jax.ShapeDtypeStruct((n,), jnp.int32),
            jax.ShapeDtypeStruct((1, 1), jnp.float32),
        ],
        scratch_shapes=[
            pltpu.VMEM((_TN,), jnp.float32),
            pltpu.VMEM((_TN,), jnp.float32),
            pltpu.VMEM((_TN,), jnp.int32),
            pltpu.VMEM((_TN,), jnp.float32),
            pltpu.VMEM((_TN,), jnp.int32),
            pltpu.VMEM((_TN,), jnp.float32),
            pltpu.SMEM((1,), jnp.float32),
        ],
    )(z, cb)
    return idx, loss11


def _sc_gather(cb, idx):
    info = plsc.get_sparse_core_info()
    nc, ns = info.num_cores, info.num_subcores
    nw = nc * ns
    n = idx.shape[0]
    d = cb.shape[1]
    b_per_w = n // nw
    ch = 256
    mesh = plsc.VectorSubcoreMesh(core_axis_name="c", subcore_axis_name="s")

    @functools.partial(
        pl.kernel, mesh=mesh,
        out_type=jax.ShapeDtypeStruct((n, d), jnp.float32),
        scratch_types=[
            pltpu.VMEM((ch,), jnp.int32),
            pltpu.VMEM((ch, d), jnp.float32),
            pltpu.SemaphoreType.DMA,
        ],
    )
    def gk(cb_hbm, idx_hbm, out_hbm, idx_v, rows_v, sem):
        wid = lax.axis_index("s") * nc + lax.axis_index("c")
        for c in range(b_per_w // ch):
            base = wid * b_per_w + c * ch
            pltpu.sync_copy(idx_hbm.at[pl.ds(base, ch)], idx_v)
            pltpu.async_copy(cb_hbm.at[idx_v], rows_v, sem).wait()
            pltpu.sync_copy(rows_v, out_hbm.at[pl.ds(base, ch)])

    return gk(cb, idx)


def kernel(motion, codebook):
    b, t, d = motion.shape
    z = motion.reshape(-1, d)
    idx, loss11 = _vq_argmin(z, codebook)
    q = _sc_gather(codebook, idx)
    return q.reshape(b, t, d), idx.reshape(b, t), loss11[0, 0]


# speedup vs baseline: 1.2476x; 1.2476x over previous
import functools

import jax
import jax.numpy as jnp
from jax import lax
from jax.experimental import pallas as pl
from jax.experimental.pallas import tpu as pltpu
from jax.experimental.pallas import tpu_sc as plsc

_TN = 256
_TK = 912
_PASS = 3


def _rn_bf16(x):
    return x.astype(jnp.bfloat16).astype(jnp.float32)


def _vq_body(scale, z_ref, cb_ref, idx_ref, loss_ref,
             zsq_ref, esq_ref, pv_ref, pi_ref, bval_ref, bidx_ref, bdist_ref,
             lsum_ref):
    i = pl.program_id(0)
    j = pl.program_id(1)
    ni = pl.num_programs(0)
    nj = pl.num_programs(1)

    z = z_ref[...]

    @pl.when(j == 0)
    def _init():
        zsq_ref[...] = jnp.sum(z * z, axis=1)
        bval_ref[...] = jnp.full((z.shape[0],), jnp.inf, jnp.float32)
        bidx_ref[...] = jnp.zeros((z.shape[0],), jnp.int32)
        bdist_ref[...] = jnp.zeros((z.shape[0],), jnp.float32)

    cb = cb_ref[...]

    tkb = cb.shape[0]

    @pl.when(i == 0)
    def _esq():
        esq_ref[pl.ds(j * tkb, tkb), :] = jnp.sum(cb * cb, axis=1,
                                                  keepdims=True)

    esq = esq_ref[pl.ds(j * tkb, tkb), :]
    mm = lax.dot_general(cb, z, (((1,), (1,)), ((), ())),
                         preferred_element_type=jnp.float32)
    dist = (zsq_ref[...][None, :] + esq) - 2.0 * mm
    m = jnp.min(dist, axis=0)
    tk = dist.shape[0]
    iota = lax.broadcasted_iota(jnp.int32, dist.shape, 0)
    gidx = jnp.min(jnp.where(dist == m[None, :], iota, tk), axis=0) + j * tk

    @pl.when(j % _PASS == 0)
    def _pass_start():
        pv_ref[...] = m
        pi_ref[...] = gidx

    @pl.when(j % _PASS != 0)
    def _pass_cont():
        upd = m < pv_ref[...]
        pi_ref[...] = jnp.where(upd, gidx, pi_ref[...])
        pv_ref[...] = jnp.where(upd, m, pv_ref[...])

    @pl.when(j % _PASS == _PASS - 1)
    def _pass_end():
        pv = pv_ref[...]
        pi = pi_ref[...]
        av = bval_ref[...]
        better = pv < av
        take = better | ((pv == av) & (pi < bidx_ref[...]))
        bidx_ref[...] = jnp.where(take, pi, bidx_ref[...])
        bdist_ref[...] = jnp.where(take, pv, bdist_ref[...])
        bval_ref[...] = jnp.where(better, _rn_bf16(pv), av)

    @pl.when(j == nj - 1)
    def _fin():
        idx_ref[...] = bidx_ref[...]

        @pl.when(i == 0)
        def _z():
            lsum_ref[0] = 0.0

        lsum_ref[0] += jnp.sum(bdist_ref[...])

        @pl.when(i == ni - 1)
        def _w():
            loss_ref[...] = jnp.full((1, 1), lsum_ref[0] * scale, jnp.float32)


def _vq_argmin(z, cb):
    n, d = z.shape
    k = cb.shape[0]
    nj = -(-k // _TK)
    kpad = nj * _TK
    if kpad != k:
        cb = jnp.concatenate(
            [cb, jnp.full((kpad - k, d), 1e4, jnp.float32)], axis=0)
    scale = 1.25 / (n * d)
    idx, loss11 = pl.pallas_call(
        functools.partial(_vq_body, scale),
        grid=(n // _TN, nj),
        in_specs=[
            pl.BlockSpec((_TN, d), lambda i, j: (i, 0)),
            pl.BlockSpec((_TK, d), lambda i, j: (j, 0)),
        ],
        out_specs=[
            pl.BlockSpec((_TN,), lambda i, j: (i,)),
            pl.BlockSpec((1, 1), lambda i, j: (0, 0)),
        ],
        out_shape=[
            jax.ShapeDtypeStruct((n,), jnp.int32),
            jax.ShapeDtypeStruct((1, 1), jnp.float32),
        ],
        scratch_shapes=[
            pltpu.VMEM((_TN,), jnp.float32),
            pltpu.VMEM((nj * _TK, 1), jnp.float32),
            pltpu.VMEM((_TN,), jnp.float32),
            pltpu.VMEM((_TN,), jnp.int32),
            pltpu.VMEM((_TN,), jnp.float32),
            pltpu.VMEM((_TN,), jnp.int32),
            pltpu.VMEM((_TN,), jnp.float32),
            pltpu.SMEM((1,), jnp.float32),
        ],
    )(z, cb)
    return idx, loss11


def _sc_gather(cb, idx):
    info = plsc.get_sparse_core_info()
    nc, ns = info.num_cores, info.num_subcores
    nw = nc * ns
    n = idx.shape[0]
    d = cb.shape[1]
    b_per_w = n // nw
    ch = 256
    mesh = plsc.VectorSubcoreMesh(core_axis_name="c", subcore_axis_name="s")

    @functools.partial(
        pl.kernel, mesh=mesh,
        out_type=jax.ShapeDtypeStruct((n, d), jnp.float32),
        scratch_types=[
            pltpu.VMEM((ch,), jnp.int32),
            pltpu.VMEM((ch, d), jnp.float32),
            pltpu.SemaphoreType.DMA,
        ],
    )
    def gk(cb_hbm, idx_hbm, out_hbm, idx_v, rows_v, sem):
        wid = lax.axis_index("s") * nc + lax.axis_index("c")
        for c in range(b_per_w // ch):
            base = wid * b_per_w + c * ch
            pltpu.sync_copy(idx_hbm.at[pl.ds(base, ch)], idx_v)
            pltpu.async_copy(cb_hbm.at[idx_v], rows_v, sem).wait()
            pltpu.sync_copy(rows_v, out_hbm.at[pl.ds(base, ch)])

    return gk(cb, idx)


def kernel(motion, codebook):
    b, t, d = motion.shape
    z = motion.reshape(-1, d)
    idx, loss11 = _vq_argmin(z, codebook)
    q = _sc_gather(codebook, idx)
    return q.reshape(b, t, d), idx.reshape(b, t), loss11[0, 0]


# speedup vs baseline: 2.3190x; 1.8588x over previous
import functools

import jax
import jax.numpy as jnp
from jax import lax
from jax.experimental import pallas as pl
from jax.experimental.pallas import tpu as pltpu
from jax.experimental.pallas import tpu_sc as plsc

_TN = 256
_PW = 2736
_CH = 304


def _rn_bf16(x):
    return x.astype(jnp.bfloat16).astype(jnp.float32)


def _vq_body(scale, z_ref, cb_ref, idx_ref, loss_ref,
             zsq_ref, esq_ref, bval_ref, bidx_ref, bdist_ref, lsum_ref):
    j = pl.program_id(0)
    i = pl.program_id(1)
    nj = pl.num_programs(0)
    ni = pl.num_programs(1)

    z = z_ref[...]

    @pl.when(j == 0)
    def _zsq():
        zsq_ref[pl.ds(i, 1), :] = jnp.sum(z * z, axis=1)[None, :]

    @pl.when(i == 0)
    def _esq():
        cb = cb_ref[...]
        esq_ref[...] = jnp.sum(cb * cb, axis=1, keepdims=True)

    zsq = zsq_ref[pl.ds(i, 1), :]

    m_run = None
    gi_run = None
    for c in range(_PW // _CH):
        cbc = cb_ref[pl.ds(c * _CH, _CH), :]
        esq_c = esq_ref[pl.ds(c * _CH, _CH), :]
        mm = lax.dot_general(cbc, z, (((1,), (1,)), ((), ())),
                             preferred_element_type=jnp.float32)
        dist = (zsq + esq_c) - 2.0 * mm
        m_c = jnp.min(dist, axis=0)
        io = lax.broadcasted_iota(jnp.int32, dist.shape, 0) \
            + (j * _PW + c * _CH)
        gi_c = jnp.min(jnp.where(dist == m_c[None, :], io, jnp.int32(2**30)),
                       axis=0)
        if m_run is None:
            m_run, gi_run = m_c, gi_c
        else:
            upd = m_c < m_run
            gi_run = jnp.where(upd, gi_c, gi_run)
            m_run = jnp.where(upd, m_c, m_run)

    @pl.when(j == 0)
    def _first():
        bval_ref[pl.ds(i, 1), :] = _rn_bf16(m_run)[None, :]
        bidx_ref[pl.ds(i, 1), :] = gi_run[None, :]
        bdist_ref[pl.ds(i, 1), :] = m_run[None, :]

    @pl.when(j != 0)
    def _merge():
        av = bval_ref[pl.ds(i, 1), :]
        bi = bidx_ref[pl.ds(i, 1), :]
        m2 = m_run[None, :]
        gi2 = gi_run[None, :]
        better = m2 < av
        take = better | ((m2 == av) & (gi2 < bi))
        bidx_ref[pl.ds(i, 1), :] = jnp.where(take, gi2, bi)
        bdist_ref[pl.ds(i, 1), :] = jnp.where(take, m2,
                                              bdist_ref[pl.ds(i, 1), :])
        bval_ref[pl.ds(i, 1), :] = jnp.where(better, _rn_bf16(m2), av)

    @pl.when(j == nj - 1)
    def _fin():
        idx_ref[...] = jnp.reshape(bidx_ref[pl.ds(i, 1), :], (-1,))

        @pl.when(i == 0)
        def _z():
            lsum_ref[0] = 0.0

        lsum_ref[0] += jnp.sum(bdist_ref[pl.ds(i, 1), :])

        @pl.when(i == ni - 1)
        def _w():
            loss_ref[...] = jnp.full((1, 1), lsum_ref[0] * scale, jnp.float32)


def _vq_argmin(z, cb):
    n, d = z.shape
    k = cb.shape[0]
    nj = -(-k // _PW)
    kpad = nj * _PW
    if kpad != k:
        cb = jnp.concatenate(
            [cb, jnp.full((kpad - k, d), 1e4, jnp.float32)], axis=0)
    ni = n // _TN
    scale = 1.25 / (n * d)
    idx, loss11 = pl.pallas_call(
        functools.partial(_vq_body, scale),
        grid=(nj, ni),
        in_specs=[
            pl.BlockSpec((_TN, d), lambda j, i: (i, 0)),
            pl.BlockSpec((_PW, d), lambda j, i: (j, 0)),
        ],
        out_specs=[
            pl.BlockSpec((_TN,), lambda j, i: (i,)),
            pl.BlockSpec((1, 1), lambda j, i: (0, 0)),
        ],
        out_shape=[
            jax.ShapeDtypeStruct((n,), jnp.int32),
            jax.ShapeDtypeStruct((1, 1), jnp.float32),
        ],
        scratch_shapes=[
            pltpu.VMEM((ni, _TN), jnp.float32),
            pltpu.VMEM((_PW, 1), jnp.float32),
            pltpu.VMEM((ni, _TN), jnp.float32),
            pltpu.VMEM((ni, _TN), jnp.int32),
            pltpu.VMEM((ni, _TN), jnp.float32),
            pltpu.SMEM((1,), jnp.float32),
        ],
    )(z, cb)
    return idx, loss11


def _sc_gather(cb, idx):
    info = plsc.get_sparse_core_info()
    nc, ns = info.num_cores, info.num_subcores
    nw = nc * ns
    n = idx.shape[0]
    d = cb.shape[1]
    b_per_w = n // nw
    ch = 256
    mesh = plsc.VectorSubcoreMesh(core_axis_name="c", subcore_axis_name="s")

    @functools.partial(
        pl.kernel, mesh=mesh,
        out_type=jax.ShapeDtypeStruct((n, d), jnp.float32),
        scratch_types=[
            pltpu.VMEM((ch,), jnp.int32),
            pltpu.VMEM((ch, d), jnp.float32),
            pltpu.SemaphoreType.DMA,
        ],
    )
    def gk(cb_hbm, idx_hbm, out_hbm, idx_v, rows_v, sem):
        wid = lax.axis_index("s") * nc + lax.axis_index("c")
        for c in range(b_per_w // ch):
            base = wid * b_per_w + c * ch
            pltpu.sync_copy(idx_hbm.at[pl.ds(base, ch)], idx_v)
            pltpu.async_copy(cb_hbm.at[idx_v], rows_v, sem).wait()
            pltpu.sync_copy(rows_v, out_hbm.at[pl.ds(base, ch)])

    return gk(cb, idx)


def kernel(motion, codebook):
    b, t, d = motion.shape
    z = motion.reshape(-1, d)
    idx, loss11 = _vq_argmin(z, codebook)
    q = _sc_gather(codebook, idx)
    return q.reshape(b, t, d), idx.reshape(b, t), loss11[0, 0]


# speedup vs baseline: 2.3953x; 1.0329x over previous
import functools

import jax
import jax.numpy as jnp
from jax import lax
from jax.experimental import pallas as pl
from jax.experimental.pallas import tpu as pltpu
from jax.experimental.pallas import tpu_sc as plsc

_TN = 256
_PW = 2736
_CH = 152


def _rn_bf16(x):
    return x.astype(jnp.bfloat16).astype(jnp.float32)


def _vq_body(scale, z_ref, cb_ref, idx_ref, loss_ref,
             zsq_ref, esq_ref, cbm2_ref, bval_ref, bidx_ref, bdist_ref,
             lsum_ref):
    j = pl.program_id(0)
    i = pl.program_id(1)
    nj = pl.num_programs(0)
    ni = pl.num_programs(1)

    z = z_ref[...]

    @pl.when(j == 0)
    def _zsq():
        zsq_ref[pl.ds(i, 1), :] = jnp.sum(z * z, axis=1)[None, :]

    @pl.when(i == 0)
    def _esq():
        cb = cb_ref[...]
        esq_ref[...] = jnp.sum(cb * cb, axis=1, keepdims=True)
        cbm2_ref[...] = cb * -2.0

    zsq = zsq_ref[pl.ds(i, 1), :]

    m_run = None
    gi_run = None
    io = lax.broadcasted_iota(jnp.int32, (_CH, _TN), 0)
    for c in range(_PW // _CH):
        cbc = cbm2_ref[pl.ds(c * _CH, _CH), :]
        esq_c = esq_ref[pl.ds(c * _CH, _CH), :]
        mm2 = lax.dot_general(cbc, z, (((1,), (1,)), ((), ())),
                              preferred_element_type=jnp.float32)
        dist = (zsq + esq_c) + mm2
        m_c = jnp.min(dist, axis=0)
        gi_c = jnp.min(jnp.where(dist == m_c[None, :], io, jnp.int32(_CH)),
                       axis=0) + (j * _PW + c * _CH)
        if m_run is None:
            m_run, gi_run = m_c, gi_c
        else:
            upd = m_c < m_run
            gi_run = jnp.where(upd, gi_c, gi_run)
            m_run = jnp.where(upd, m_c, m_run)

    @pl.when(j == 0)
    def _first():
        bval_ref[pl.ds(i, 1), :] = _rn_bf16(m_run)[None, :]
        bidx_ref[pl.ds(i, 1), :] = gi_run[None, :]
        bdist_ref[pl.ds(i, 1), :] = m_run[None, :]

    @pl.when(j != 0)
    def _merge():
        av = bval_ref[pl.ds(i, 1), :]
        bi = bidx_ref[pl.ds(i, 1), :]
        m2 = m_run[None, :]
        gi2 = gi_run[None, :]
        better = m2 < av
        take = better | ((m2 == av) & (gi2 < bi))
        bidx_ref[pl.ds(i, 1), :] = jnp.where(take, gi2, bi)
        bdist_ref[pl.ds(i, 1), :] = jnp.where(take, m2,
                                              bdist_ref[pl.ds(i, 1), :])
        bval_ref[pl.ds(i, 1), :] = jnp.where(better, _rn_bf16(m2), av)

    @pl.when(j == nj - 1)
    def _fin():
        idx_ref[...] = jnp.reshape(bidx_ref[pl.ds(i, 1), :], (-1,))

        @pl.when(i == 0)
        def _z():
            lsum_ref[0] = 0.0

        lsum_ref[0] += jnp.sum(bdist_ref[pl.ds(i, 1), :])

        @pl.when(i == ni - 1)
        def _w():
            loss_ref[...] = jnp.full((1, 1), lsum_ref[0] * scale, jnp.float32)


def _vq_argmin(z, cb):
    n, d = z.shape
    k = cb.shape[0]
    nj = -(-k // _PW)
    kpad = nj * _PW
    if kpad != k:
        cb = jnp.concatenate(
            [cb, jnp.full((kpad - k, d), 1e4, jnp.float32)], axis=0)
    ni = n // _TN
    scale = 1.25 / (n * d)
    idx, loss11 = pl.pallas_call(
        functools.partial(_vq_body, scale),
        grid=(nj, ni),
        in_specs=[
            pl.BlockSpec((_TN, d), lambda j, i: (i, 0)),
            pl.BlockSpec((_PW, d), lambda j, i: (j, 0)),
        ],
        out_specs=[
            pl.BlockSpec((_TN,), lambda j, i: (i,)),
            pl.BlockSpec((1, 1), lambda j, i: (0, 0)),
        ],
        out_shape=[
            jax.ShapeDtypeStruct((n,), jnp.int32),
            jax.ShapeDtypeStruct((1, 1), jnp.float32),
        ],
        scratch_shapes=[
            pltpu.VMEM((ni, _TN), jnp.float32),
            pltpu.VMEM((_PW, 1), jnp.float32),
            pltpu.VMEM((_PW, d), jnp.float32),
            pltpu.VMEM((ni, _TN), jnp.float32),
            pltpu.VMEM((ni, _TN), jnp.int32),
            pltpu.VMEM((ni, _TN), jnp.float32),
            pltpu.SMEM((1,), jnp.float32),
        ],
    )(z, cb)
    return idx, loss11


def _sc_gather(cb, idx):
    info = plsc.get_sparse_core_info()
    nc, ns = info.num_cores, info.num_subcores
    nw = nc * ns
    n = idx.shape[0]
    d = cb.shape[1]
    b_per_w = n // nw
    ch = 256
    mesh = plsc.VectorSubcoreMesh(core_axis_name="c", subcore_axis_name="s")

    @functools.partial(
        pl.kernel, mesh=mesh,
        out_type=jax.ShapeDtypeStruct((n, d), jnp.float32),
        scratch_types=[
            pltpu.VMEM((ch,), jnp.int32),
            pltpu.VMEM((ch, d), jnp.float32),
            pltpu.SemaphoreType.DMA,
        ],
    )
    def gk(cb_hbm, idx_hbm, out_hbm, idx_v, rows_v, sem):
        wid = lax.axis_index("s") * nc + lax.axis_index("c")
        for c in range(b_per_w // ch):
            base = wid * b_per_w + c * ch
            pltpu.sync_copy(idx_hbm.at[pl.ds(base, ch)], idx_v)
            pltpu.async_copy(cb_hbm.at[idx_v], rows_v, sem).wait()
            pltpu.sync_copy(rows_v, out_hbm.at[pl.ds(base, ch)])

    return gk(cb, idx)


def kernel(motion, codebook):
    b, t, d = motion.shape
    z = motion.reshape(-1, d)
    idx, loss11 = _vq_argmin(z, codebook)
    q = _sc_gather(codebook, idx)
    return q.reshape(b, t, d), idx.reshape(b, t), loss11[0, 0]
